# Initial kernel scaffold; baseline (speedup 1.0000x reference)
import jax
import jax.numpy as jnp
from jax.experimental import pallas as pl


def kernel(x, table):
    raise NotImplementedError("write your pallas kernel here")


# speedup vs baseline: 1.8714x; 1.8714x over previous
import functools

import jax
import jax.numpy as jnp
from jax import lax
from jax.experimental import pallas as pl
from jax.experimental.pallas import tpu as pltpu
from jax.experimental.pallas import tpu_sc as plsc

NC = 2
NS = 16
NW = NC * NS

B = 16384 * 50
D = 64
BPW = B // NW
C = 512
NG = BPW // C

_mesh = plsc.VectorSubcoreMesh(
    core_axis_name="c", subcore_axis_name="s", num_cores=NC, num_subcores=NS
)


@functools.partial(
    pl.kernel,
    out_type=jax.ShapeDtypeStruct((B, D), jnp.float32),
    mesh=_mesh,
    compiler_params=pltpu.CompilerParams(use_tc_tiling_on_sc=False),
    scratch_types=[
        pltpu.VMEM((BPW,), jnp.int32),
        pltpu.VMEM((C, D), jnp.float32),
        pltpu.VMEM((C, D), jnp.float32),
        pltpu.SemaphoreType.DMA,
        pltpu.SemaphoreType.DMA,
    ],
)
def _embed_gather(idx_hbm, table_hbm, out_hbm, idx_v, buf0, buf1, g0, g1):
    wid = lax.axis_index("s") * NC + lax.axis_index("c")
    base = wid * BPW

    pltpu.sync_copy(idx_hbm.at[pl.ds(base, BPW)], idx_v)

    def fire(group, buf, sem):
        off = pl.multiple_of(group * C, C)
        pltpu.async_copy(table_hbm.at[idx_v.at[pl.ds(off, C)]], buf, sem)

    def drain(buf, sem):
        pltpu.make_async_copy(table_hbm.at[pl.ds(0, C)], buf, sem).wait()

    def write(group, buf):
        row = base + pl.multiple_of(group * C, C)
        pltpu.sync_copy(buf, out_hbm.at[pl.ds(row, C)])

    fire(0, buf0, g0)

    def body(i, _):
        ga = 2 * i
        fire(ga + 1, buf1, g1)
        drain(buf0, g0)
        write(ga, buf0)
        gb = jnp.minimum(ga + 2, 2 * (NG // 2) - 2)
        fire(gb, buf0, g0)
        drain(buf1, g1)
        write(ga + 1, buf1)
        return 0

    lax.fori_loop(0, NG // 2, body, 0)
    drain(buf0, g0)


def kernel(x, table):
    flat = x.reshape(-1).astype(jnp.int32)
    out = _embed_gather(flat, table)
    return out.reshape(x.shape + (D,))


# speedup vs baseline: 1.9567x; 1.0456x over previous
import functools

import jax
import jax.numpy as jnp
from jax import lax
from jax.experimental import pallas as pl
from jax.experimental.pallas import tpu as pltpu
from jax.experimental.pallas import tpu_sc as plsc

NC = 2
NS = 16
NW = NC * NS

B = 16384 * 50
D = 64
BPW = B // NW
C = 512
NG = BPW // C

_mesh = plsc.VectorSubcoreMesh(
    core_axis_name="c", subcore_axis_name="s", num_cores=NC, num_subcores=NS
)


@functools.partial(
    pl.kernel,
    out_type=jax.ShapeDtypeStruct((B, D), jnp.float32),
    mesh=_mesh,
    compiler_params=pltpu.CompilerParams(use_tc_tiling_on_sc=False),
    scratch_types=[
        pltpu.VMEM((BPW,), jnp.int32),
        pltpu.VMEM((C, D), jnp.float32),
        pltpu.VMEM((C, D), jnp.float32),
        pltpu.SemaphoreType.DMA,
        pltpu.SemaphoreType.DMA,
    ],
)
def _embed_gather(idx_hbm, table_hbm, out_hbm, idx_v, buf0, buf1, g0, g1):
    wid = lax.axis_index("s") * NC + lax.axis_index("c")
    base = wid * BPW

    pltpu.sync_copy(idx_hbm.at[pl.ds(base, BPW)], idx_v)

    def fire(group, buf, sem):
        off = pl.multiple_of(group * C, C)
        pltpu.async_copy(table_hbm.at[idx_v.at[pl.ds(off, C)]], buf, sem)

    def drain(buf, sem):
        pltpu.make_async_copy(table_hbm.at[pl.ds(0, C)], buf, sem).wait()

    def write(group, buf):
        row = base + pl.multiple_of(group * C, C)
        pltpu.sync_copy(buf, out_hbm.at[pl.ds(row, C)])

    fire(0, buf0, g0)

    def body(i, _):
        ga = 2 * i
        fire(ga + 1, buf1, g1)
        drain(buf0, g0)
        write(ga, buf0)
        gb = jnp.minimum(ga + 2, 2 * (NG // 2) - 2)
        fire(gb, buf0, g0)
        drain(buf1, g1)
        write(ga + 1, buf1)
        return 0

    lax.fori_loop(0, NG // 2, body, 0)
    drain(buf0, g0)


def kernel(x, table):
    flat = x.T.reshape(-1).astype(jnp.int32)
    out = _embed_gather(flat, table)
    s, b = x.shape[1], x.shape[0]
    return out.reshape(s, b, D).transpose(1, 0, 2)
